# baseline (speedup 1.0000x reference)
import jax
import jax.numpy as jnp
from jax import lax
from jax.experimental import pallas as pl
from jax.experimental.pallas import tpu as pltpu
from jax.experimental.pallas import tpu_sc as plsc

B = 16
Z = 9
CH = 256
X = 32
Y = 32
CHANNELS = 86
L = 16

_OFF_COL = 0
_OFF_ROW = X * CHANNELS
_OFF_HEI = 2 * X * CHANNELS
_HEI_PAD = 776
_W_SIZE = _OFF_HEI + _HEI_PAD


def _sc_body(col_hbm, row_hbm, hei_hbm, out_hbm, w_v, buf, sem):
    nc = 2
    wid = lax.axis_index("s") * nc + lax.axis_index("c")

    pltpu.sync_copy(col_hbm, w_v.at[pl.ds(_OFF_COL, _OFF_ROW)])
    pltpu.sync_copy(row_hbm, w_v.at[pl.ds(_OFF_ROW, _OFF_ROW)])
    pltpu.sync_copy(hei_hbm, w_v.at[pl.ds(_OFF_HEI, _HEI_PAD)])

    lane = lax.iota(jnp.int32, L)
    row_base = _OFF_ROW - CHANNELS + wid * CHANNELS

    for t in range(Z):
        hei_base = _OFF_HEI - 2 * CHANNELS + t * CHANNELS

        def _j_body(j, _, t=t, hei_base=hei_base):
            col_base = j * CHANNELS
            row = t * 32 + j
            for lg in range(CH // L):
                c0 = lg * L
                cvec = lane + c0
                if c0 + L <= CHANNELS:
                    v = w_v[pl.ds(col_base + c0, L)]
                elif c0 >= CHANNELS and c0 + L <= 2 * CHANNELS:
                    v = w_v[pl.ds(row_base + c0, L)]
                elif c0 >= 2 * CHANNELS:
                    v = w_v[pl.ds(hei_base + c0, L)]
                elif c0 < CHANNELS:
                    va = w_v[pl.ds(col_base + c0, L)]
                    vb = w_v[pl.ds(row_base + c0, L)]
                    v = jnp.where(cvec < CHANNELS, va, vb)
                else:
                    va = w_v[pl.ds(row_base + c0, L)]
                    vb = w_v[pl.ds(hei_base + c0, L)]
                    v = jnp.where(cvec < 2 * CHANNELS, va, vb)
                buf[row, pl.ds(c0, L)] = v
            return 0

        lax.fori_loop(0, X, _j_body, 0)

    copies = []
    for bb in range(B):
        for t in range(Z):
            copies.append(pltpu.async_copy(
                buf.at[pl.ds(t * 32, 32), :],
                out_hbm.at[bb, t, pl.ds(wid * 32, 32), :],
                sem,
            ))
    for cp in copies:
        cp.wait()


def kernel(tensor, row_w, col_w, hei_w):
    del tensor
    mesh = plsc.VectorSubcoreMesh(core_axis_name="c", subcore_axis_name="s")
    run = pl.kernel(
        _sc_body,
        out_type=jax.ShapeDtypeStruct((B, Z, X * Y, CH), jnp.float32),
        mesh=mesh,
        scratch_types=[
            pltpu.VMEM((_W_SIZE,), jnp.float32),
            pltpu.VMEM((Z * 32, CH), jnp.float32),
            pltpu.SemaphoreType.DMA,
        ],
    )
    out = run(
        col_w.reshape(-1),
        row_w.reshape(-1),
        jnp.pad(hei_w.reshape(-1), (0, _HEI_PAD - Z * CHANNELS)),
    )
    return out.reshape(B, Z, X, Y, CH).transpose(0, 1, 4, 2, 3)


# speedup vs baseline: 1.6841x; 1.6841x over previous
import jax
import jax.numpy as jnp
from jax.experimental import pallas as pl
from jax.experimental.pallas import tpu as pltpu

B = 16
Z = 9
CH = 256
X = 32
Y = 32
CHANNELS = 86


def _pos_body(col_ref, row_ref, hei_ref, out_ref, scratch, sem):
    cw = col_ref[...]
    rw = row_ref[...]
    hw = hei_ref[...]

    ci = jax.lax.broadcasted_iota(jnp.int32, (X, Y, CH), 2)
    a = jnp.broadcast_to(cw[None, :, :], (X, Y, CH))
    b = jnp.broadcast_to(rw[:, None, :], (X, Y, CH))
    ab = jnp.where(ci < CHANNELS, a, b)
    for z in range(Z):
        c = jnp.broadcast_to(hw[z][None, None, :], (X, Y, CH))
        scratch[z] = jnp.where(ci < 2 * CHANNELS, ab, c)

    copies = [
        pltpu.make_async_copy(scratch, out_ref.at[bb], sem)
        for bb in range(B)
    ]
    for cp in copies:
        cp.start()
    for cp in copies:
        cp.wait()


def kernel(tensor, row_w, col_w, hei_w):
    del tensor
    cw256 = jnp.pad(col_w, ((0, 0), (0, CH - CHANNELS)))
    rw256 = jnp.pad(row_w, ((0, 0), (CHANNELS, CH - 2 * CHANNELS)))
    hei256 = jnp.pad(hei_w[:, : CH - 2 * CHANNELS], ((0, 0), (2 * CHANNELS, 0)))
    out = pl.pallas_call(
        _pos_body,
        grid=(1,),
        in_specs=[
            pl.BlockSpec((Y, CH), lambda g: (0, 0)),
            pl.BlockSpec((X, CH), lambda g: (0, 0)),
            pl.BlockSpec((Z, CH), lambda g: (0, 0)),
        ],
        out_specs=pl.BlockSpec(memory_space=pl.ANY),
        out_shape=jax.ShapeDtypeStruct((B, Z, X, Y, CH), jnp.float32),
        scratch_shapes=[
            pltpu.VMEM((Z, X, Y, CH), jnp.float32),
            pltpu.SemaphoreType.DMA,
        ],
        compiler_params=pltpu.CompilerParams(
            dimension_semantics=("arbitrary",),
        ),
    )(cw256, rw256, hei256)
    return jnp.transpose(out, (0, 1, 4, 2, 3))


# speedup vs baseline: 1.8743x; 1.1130x over previous
import jax
import jax.numpy as jnp
from jax.experimental import pallas as pl
from jax.experimental.pallas import tpu as pltpu

B = 16
Z = 9
CH = 256
X = 32
Y = 32
CHANNELS = 86
C_REST = CH - 2 * CHANNELS


def _pos_body(col_ref, row_ref, hei_ref, out_ref, scratch, sem):
    z = pl.program_id(0)
    zeros = jnp.zeros((X, CHANNELS), jnp.float32)
    cw = jnp.concatenate(
        [col_ref[...], zeros, zeros[:, : CH - 2 * CHANNELS]], axis=1)
    rw = jnp.concatenate(
        [zeros, row_ref[...], zeros[:, : CH - 2 * CHANNELS]], axis=1)
    hz = hei_ref[pl.ds(z, 1), :]
    hw = jnp.concatenate(
        [jnp.zeros((1, 2 * CHANNELS), jnp.float32), hz[:, :C_REST]], axis=1)

    ci = jax.lax.broadcasted_iota(jnp.int32, (X, Y, CH), 2)
    a = jnp.broadcast_to(cw[None, :, :], (X, Y, CH))
    b = jnp.broadcast_to(rw[:, None, :], (X, Y, CH))
    ab = jnp.where(ci < CHANNELS, a, b)
    c = jnp.broadcast_to(hw[0][None, None, :], (X, Y, CH))
    scratch[pl.ds(z, 1)] = jnp.where(ci < 2 * CHANNELS, ab, c)[None]

    def _copies(zz):
        return [
            pltpu.make_async_copy(
                scratch.at[pl.ds(zz, 1)],
                out_ref.at[bb, pl.ds(zz, 1)],
                sem,
            )
            for bb in range(B)
        ]

    for cp in _copies(z):
        cp.start()

    @pl.when(z > 0)
    def _():
        for cp in _copies(z - 1):
            cp.wait()

    @pl.when(z == Z - 1)
    def _():
        for cp in _copies(z):
            cp.wait()


def kernel(tensor, row_w, col_w, hei_w):
    del tensor
    out = pl.pallas_call(
        _pos_body,
        grid=(Z,),
        in_specs=[
            pl.BlockSpec((Y, CHANNELS), lambda z: (0, 0)),
            pl.BlockSpec((X, CHANNELS), lambda z: (0, 0)),
            pl.BlockSpec((Z, CHANNELS), lambda z: (0, 0)),
        ],
        out_specs=pl.BlockSpec(memory_space=pl.ANY),
        out_shape=jax.ShapeDtypeStruct((B, Z, X, Y, CH), jnp.float32),
        scratch_shapes=[
            pltpu.VMEM((Z, X, Y, CH), jnp.float32),
            pltpu.SemaphoreType.DMA,
        ],
        compiler_params=pltpu.CompilerParams(
            dimension_semantics=("arbitrary",),
        ),
    )(col_w, row_w, hei_w)
    return jnp.transpose(out, (0, 1, 4, 2, 3))


# speedup vs baseline: 1.8772x; 1.0015x over previous
import jax
import jax.numpy as jnp
from jax.experimental import pallas as pl
from jax.experimental.pallas import tpu as pltpu

B = 16
Z = 9
CH = 256
X = 32
Y = 32
CHANNELS = 86
C_REST = CH - 2 * CHANNELS


def _pos_body(col_ref, row_ref, hei_ref, out_ref, scratch, cr_s, sem):
    z = pl.program_id(0)
    ci = jax.lax.broadcasted_iota(jnp.int32, (X, Y, CH), 2)

    @pl.when(z == 0)
    def _():
        zeros = jnp.zeros((X, CHANNELS), jnp.float32)
        cw = jnp.concatenate(
            [col_ref[...], zeros, zeros[:, :C_REST]], axis=1)
        rw = jnp.concatenate(
            [zeros, row_ref[...], zeros[:, :C_REST]], axis=1)
        a = jnp.broadcast_to(cw[None, :, :], (X, Y, CH))
        b = jnp.broadcast_to(rw[:, None, :], (X, Y, CH))
        cr_s[...] = jnp.where(ci < CHANNELS, a, b)

    hz = hei_ref[pl.ds(z, 1), :]
    hw = jnp.concatenate(
        [jnp.zeros((1, 2 * CHANNELS), jnp.float32), hz[:, :C_REST]], axis=1)
    c = jnp.broadcast_to(hw[0][None, None, :], (X, Y, CH))
    scratch[pl.ds(z, 1)] = jnp.where(ci < 2 * CHANNELS, cr_s[...], c)[None]

    def _copies(zz):
        return [
            pltpu.make_async_copy(
                scratch.at[pl.ds(zz, 1)],
                out_ref.at[bb, pl.ds(zz, 1)],
                sem,
            )
            for bb in range(B)
        ]

    for cp in _copies(z):
        cp.start()

    @pl.when(z > 0)
    def _():
        for cp in _copies(z - 1):
            cp.wait()

    @pl.when(z == Z - 1)
    def _():
        for cp in _copies(z):
            cp.wait()


def kernel(tensor, row_w, col_w, hei_w):
    del tensor
    out = pl.pallas_call(
        _pos_body,
        grid=(Z,),
        in_specs=[
            pl.BlockSpec((Y, CHANNELS), lambda z: (0, 0)),
            pl.BlockSpec((X, CHANNELS), lambda z: (0, 0)),
            pl.BlockSpec((Z, CHANNELS), lambda z: (0, 0)),
        ],
        out_specs=pl.BlockSpec(memory_space=pl.ANY),
        out_shape=jax.ShapeDtypeStruct((B, Z, X, Y, CH), jnp.float32),
        scratch_shapes=[
            pltpu.VMEM((Z, X, Y, CH), jnp.float32),
            pltpu.VMEM((X, Y, CH), jnp.float32),
            pltpu.SemaphoreType.DMA,
        ],
        compiler_params=pltpu.CompilerParams(
            dimension_semantics=("arbitrary",),
        ),
    )(col_w, row_w, hei_w)
    return jnp.transpose(out, (0, 1, 4, 2, 3))
